# baseline (speedup 1.0000x reference)
import functools

import jax
import jax.numpy as jnp
from jax import lax
from jax.experimental import pallas as pl
from jax.experimental.pallas import tpu as pltpu
from jax.experimental.pallas import tpu_sc as plsc

N, E, DF, DE, HD = 10000, 320000, 128, 16, 32
NT = 16
NW = 2 * NT
EPT = E // NT
EPW = E // NW
DCH = 2000
C = 80
NCH = EPW // C
NBUF = 5
GRP = NCH // NBUF
NPAD = 10240
NR = NPAD // 16
G2 = 2 * HD
E8 = E // 8


def _tc_nodes_body(x_ref, w_ref, b_ref, u_ref, b0_ref):
    ub = jnp.dot(x_ref[...], w_ref[...], preferred_element_type=jnp.float32)
    u_ref[...] = ub[:, :G2]
    b0_ref[...] = ub[:, G2:] + b_ref[0][None, :]


def _tc_nodes(xg, w, b):
    return pl.pallas_call(
        _tc_nodes_body,
        in_specs=[
            pl.BlockSpec((N, DF), lambda: (0, 0)),
            pl.BlockSpec((DF, 2 * G2), lambda: (0, 0)),
            pl.BlockSpec((1, G2), lambda: (0, 0)),
        ],
        out_specs=[
            pl.BlockSpec((N, G2), lambda: (0, 0)),
            pl.BlockSpec((N, G2), lambda: (0, 0)),
        ],
        out_shape=[
            jax.ShapeDtypeStruct((N, G2), jnp.float32),
            jax.ShapeDtypeStruct((N, G2), jnp.float32),
        ],
    )(xg, w, b)


EBLK8 = E8 // 5


def _tc_edges_body(xe_ref, w1_ref, b1_ref, w2_ref, b2_ref, v_ref):
    h = jnp.maximum(
        jnp.dot(xe_ref[...], w1_ref[...], preferred_element_type=jnp.float32)
        + b1_ref[0][None, :], 0.0)
    v_ref[...] = jnp.dot(h, w2_ref[...],
                         preferred_element_type=jnp.float32) + b2_ref[0]


def _tc_edges(xep, w1b, b1b, w2b, b2):
    return pl.pallas_call(
        _tc_edges_body,
        grid=(5,),
        in_specs=[
            pl.BlockSpec((EBLK8, 8 * DE), lambda j: (j, 0)),
            pl.BlockSpec((8 * DE, 8 * HD), lambda j: (0, 0)),
            pl.BlockSpec((1, 8 * HD), lambda j: (0, 0)),
            pl.BlockSpec((8 * HD, 8), lambda j: (0, 0)),
            pl.BlockSpec(memory_space=pltpu.SMEM),
        ],
        out_specs=pl.BlockSpec((EBLK8, 8), lambda j: (j, 0)),
        out_shape=jax.ShapeDtypeStruct((E8, 8), jnp.float32),
    )(xep, w1b, b1b, w2b, b2)


def _sc_body(idx_hbm, v_hbm, u_hbm, s_hbm,
             s_sh, deg_sh,
             drow, dew, row2, lw2, deg2d, zdeg, i128,
             *bufs):
    rows = (bufs[0:NBUF], bufs[NBUF:2 * NBUF])
    colc = (bufs[2 * NBUF:3 * NBUF], bufs[3 * NBUF:4 * NBUF])
    gsem = bufs[4 * NBUF:4 * NBUF + 2]
    ssem = bufs[4 * NBUF + 2:4 * NBUF + 4]
    cid = lax.axis_index("c")
    sid = lax.axis_index("s")
    wid = cid * NT + sid
    zf16 = jnp.zeros((16,), jnp.float32)

    def _z1(i, _):
        deg2d[i] = zf16
        return 0
    lax.fori_loop(0, NR, _z1, 0)

    def _zz(i, _):
        zdeg[i] = zf16
        return 0
    lax.fori_loop(0, NR // NT, _zz, 0)

    def _z2(i, _):
        for j in range(G2 // 16):
            rows[0][0][i, pl.ds(j * 16, 16)] = zf16
        return 0
    lax.fori_loop(0, C, _z2, 0)

    for q in range(NPAD // NT // C):
        pltpu.sync_copy(rows[0][0],
                        s_sh.at[pl.ds(sid * (NPAD // NT) + q * C, C)])
    pltpu.sync_copy(zdeg, deg_sh.at[pl.ds(sid * (NR // NT), NR // NT)])

    pltpu.sync_copy(idx_hbm.at[0, pl.ds(wid * EPW, EPW)], row2)
    pltpu.sync_copy(v_hbm.at[pl.ds(wid * EPW, EPW)], lw2)

    def _sigmoid16(xv):
        t = jnp.exp(-jnp.maximum(xv, -80.0)) + 1.0
        ib = jnp.int32(0x7EF311C3) - lax.bitcast_convert_type(t, jnp.int32)
        r = lax.bitcast_convert_type(ib, jnp.float32)
        r = r * (2.0 - t * r)
        r = r * (2.0 - t * r)
        r = r * (2.0 - t * r)
        return r

    def _sig(i, _):
        o = pl.ds(i * 16, 16)
        lw2[o] = _sigmoid16(lw2[o])
        return 0
    lax.fori_loop(0, EPW // 16, _sig, 0)

    def _dchunk(q, _):
        base = sid * EPT + q * DCH
        pltpu.sync_copy(idx_hbm.at[0, pl.ds(base, DCH)], drow)
        pltpu.sync_copy(v_hbm.at[pl.ds(base, DCH)], dew)

        def _dacc(i, _):
            o = pl.ds(i * 16, 16)
            r = drow[o]
            e = _sigmoid16(dew[o])
            plsc.addupdate_scatter(
                deg2d, [lax.shift_right_logical(r, 4),
                        jnp.bitwise_and(r, 15)], e)
            return 0
        lax.fori_loop(0, DCH // 16, _dacc, 0)
        return 0
    lax.fori_loop(0, EPT // DCH, _dchunk, 0)

    plsc.subcore_barrier()
    iota16 = lax.iota(jnp.int32, 16)
    for q in range(NR // 128):
        for j in range(8):
            i128[pl.ds(j * 16, 16)] = iota16 + (q * 128 + j * 16)
        pltpu.sync_copy(deg2d.at[pl.ds(q * 128, 128)],
                        deg_sh.at[i128], add=True)
    plsc.subcore_barrier()
    pltpu.sync_copy(deg_sh, deg2d)

    def _dinv(i, _):
        d = deg2d[i]
        ib = jnp.int32(0x5F3759DF) - lax.shift_right_arithmetic(
            lax.bitcast_convert_type(d, jnp.int32), 1)
        y = lax.bitcast_convert_type(ib, jnp.float32)
        y = y * (1.5 - 0.5 * d * y * y)
        y = y * (1.5 - 0.5 * d * y * y)
        y = y * (1.5 - 0.5 * d * y * y)
        deg2d[i] = jnp.where(d > 0.0, y, 0.0)
        return 0
    lax.fori_loop(0, NR, _dinv, 0)

    def _gather(g, st, b):
        ch = (g * NBUF + b) * C
        pltpu.async_copy(u_hbm.at[row2.at[pl.ds(ch, C)]],
                         rows[st][b], gsem[st])
        pltpu.async_copy(idx_hbm.at[1, pl.ds(wid * EPW + ch, C)],
                         colc[st][b], gsem[st])

    def _gdrain(st, b):
        pltpu.make_async_copy(
            u_hbm.at[pl.ds(0, C)], rows[st][b], gsem[st]).wait()
        pltpu.make_async_copy(
            idx_hbm.at[0, pl.ds(0, C)], colc[st][b], gsem[st]).wait()

    def _sdrain(st, b):
        pltpu.make_async_copy(
            u_hbm.at[pl.ds(0, C)], rows[st][b], ssem[st]).wait()

    for b in range(NBUF):
        _gather(0, 0, b)

    def _process(g, st):
        for b in range(NBUF):
            _gdrain(st, b)

        @pl.when(g > 0)
        def _():
            for b in range(NBUF):
                _sdrain(1 - st, b)

        @pl.when(g + 1 < GRP)
        def _():
            for b in range(NBUF):
                _gather(g + 1, 1 - st, b)

        for b in range(NBUF):
            ch = g * NBUF + b

            def _scale(m, _):
                o16 = pl.ds(ch * C + m * 16, 16)
                r = row2[o16]
                c = colc[st][b][pl.ds(m * 16, 16)]
                e = lw2[o16]
                a = plsc.load_gather(
                    deg2d, [lax.shift_right_logical(r, 4),
                            jnp.bitwise_and(r, 15)])
                bb = plsc.load_gather(
                    deg2d, [lax.shift_right_logical(c, 4),
                            jnp.bitwise_and(c, 15)])
                lv = -(a * e) * bb - jnp.where(r == c, 1.0, 0.0)
                for t in range(16):
                    ei = m * 16 + t
                    for j in range(G2 // 16):
                        o = pl.ds(j * 16, 16)
                        rows[st][b][ei, o] = rows[st][b][ei, o] * lv[t]
                return 0
            lax.fori_loop(0, C // 16, _scale, 0)
            pltpu.async_copy(rows[st][b], s_sh.at[colc[st][b]],
                             ssem[st], add=True)

    def _grp(gp, _):
        _process(2 * gp, 0)
        _process(2 * gp + 1, 1)
        return 0
    lax.fori_loop(0, GRP // 2, _grp, 0)
    _process(GRP - 1, 0)
    for b in range(NBUF):
        _sdrain(0, b)

    plsc.subcore_barrier()
    pltpu.sync_copy(s_sh.at[pl.ds(sid * (NPAD // NT), NPAD // NT)],
                    s_hbm.at[cid, pl.ds(sid * (NPAD // NT), NPAD // NT)])


@functools.cache
def _make_sc_spmm():
    @functools.partial(
        pl.kernel,
        out_type=jax.ShapeDtypeStruct((2, NPAD, G2), jnp.float32),
        mesh=plsc.VectorSubcoreMesh(core_axis_name="c", subcore_axis_name="s"),
        compiler_params=pltpu.CompilerParams(
            needs_layout_passes=False, use_tc_tiling_on_sc=False),
        scratch_types=[
            pltpu.VMEM_SHARED((NPAD, G2), jnp.float32),
            pltpu.VMEM_SHARED((NR, 16), jnp.float32),
            pltpu.VMEM((DCH,), jnp.int32),
            pltpu.VMEM((DCH,), jnp.float32),
            pltpu.VMEM((EPW,), jnp.int32),
            pltpu.VMEM((EPW,), jnp.float32),
            pltpu.VMEM((NR, 16), jnp.float32),
            pltpu.VMEM((NR // NT, 16), jnp.float32),
            pltpu.VMEM((128,), jnp.int32),
        ] + [pltpu.VMEM((C, G2), jnp.float32)] * (2 * NBUF)
          + [pltpu.VMEM((C,), jnp.int32)] * (2 * NBUF)
          + [pltpu.SemaphoreType.DMA] * 4,
    )
    def _sc_spmm(idx_hbm, v_hbm, u_hbm, s_hbm, *rest):
        _sc_body(idx_hbm, v_hbm, u_hbm, s_hbm, *rest)
    return _sc_spmm


KR = N * HD // 128
KRP = NPAD * HD // 128


def _tc_post_body(bz_ref, bh_ref, s0z_ref, s0h_ref, s1z_ref, s1h_ref,
                  med_ref, key_ref):
    az = bz_ref[...] + s0z_ref[:KR] + s1z_ref[:KR]
    ah = bh_ref[...] + s0h_ref[:KR] + s1h_ref[:KR]
    tz = jnp.tanh(az * 0.5)
    o = (0.5 - 0.5 * tz) * jnp.tanh(ah)
    b = lax.bitcast_convert_type(o, jnp.uint32)
    key_ref[...] = jnp.where((b >> 31) != 0, ~b, b | jnp.uint32(0x80000000))

    kth = (N * HD - 1) // 2

    def _bit(i, carry):
        prefix, k = carry
        sh = 31 - i
        cnt0 = jnp.sum(((key_ref[...] >> sh) == (prefix >> sh))
                       .astype(jnp.int32))
        take1 = k >= cnt0
        prefix = jnp.where(take1, prefix | (jnp.uint32(1) << sh), prefix)
        k = jnp.where(take1, k - cnt0, k)
        return prefix, k

    prefix, _ = lax.fori_loop(
        0, 32, _bit, (jnp.uint32(0), jnp.int32(kth)))
    medbits = jnp.where(
        (prefix >> 31) != 0, prefix ^ jnp.uint32(0x80000000), ~prefix)
    med_ref[0] = lax.bitcast_convert_type(medbits, jnp.float32)


def _tc_post(b0, s):
    bz = b0[:, :HD].reshape(KR, 128)
    bh = b0[:, HD:].reshape(KR, 128)
    s0z = s[0, :, :HD].reshape(KRP, 128)
    s0h = s[0, :, HD:].reshape(KRP, 128)
    s1z = s[1, :, :HD].reshape(KRP, 128)
    s1h = s[1, :, HD:].reshape(KRP, 128)
    full = lambda r: pl.BlockSpec((r, 128), lambda: (0, 0))
    return pl.pallas_call(
        _tc_post_body,
        in_specs=[full(KR), full(KR), full(KRP), full(KRP),
                  full(KRP), full(KRP)],
        out_specs=pl.BlockSpec(memory_space=pltpu.SMEM),
        out_shape=jax.ShapeDtypeStruct((1,), jnp.float32),
        scratch_shapes=[pltpu.VMEM((KR, 128), jnp.uint32)],
    )(bz, bh, s0z, s0h, s1z, s1h)


def _prep(xg, xeg, p):
    w = jnp.concatenate(
        [p["xz_w1"], p["xh_w1"], p["xz_w0"], p["xh_w0"]], 1)
    b = jnp.concatenate(
        [p["xz_b"] + p["hz_b"], p["xh_b"] + p["hh_b"]])[None, :]
    u, b0 = _tc_nodes(xg, w, b)
    eye8 = jnp.eye(8, dtype=jnp.float32)
    v = _tc_edges(
        xeg.reshape(E8, 8 * DE),
        jnp.kron(eye8, p["seq_w1"]),
        jnp.tile(p["seq_b1"], 8)[None, :],
        jnp.kron(eye8, p["seq_w2"]),
        p["seq_b2"])
    return u, b0, v.reshape(E)


def kernel(x, xi, xe, y, yi, ye, p1, p2):
    u1, b01, v1 = _prep(x, xe, p1)
    v1, u1, y2, ye2 = lax.optimization_barrier((v1, u1, y, ye))
    s1 = _make_sc_spmm()(xi, v1, u1)
    u2, b02, v2 = _prep(y2, ye2, p2)
    s2 = _make_sc_spmm()(yi, v2, u2)
    med1 = _tc_post(b01, s1)
    med2 = _tc_post(b02, s2)
    return jax.nn.sigmoid(med1[0] - med2[0])


# speedup vs baseline: 1.1498x; 1.1498x over previous
import functools

import jax
import jax.numpy as jnp
from jax import lax
from jax.experimental import pallas as pl
from jax.experimental.pallas import tpu as pltpu
from jax.experimental.pallas import tpu_sc as plsc

N, E, DF, DE, HD = 10000, 320000, 128, 16, 32
NT = 16
NW = 2 * NT
EPT = E // NT
EPW = E // NW
DCH = 800
C = 80
NCH = EPW // C
NBUF = 5
GRP = NCH // NBUF
NPAD = 10240
NR = NPAD // 16
G2 = 2 * HD
E8 = E // 8


def _tc_nodes_body(x_ref, w_ref, b_ref, u_ref, b0_ref):
    ub = jnp.dot(x_ref[...], w_ref[...], preferred_element_type=jnp.float32)
    u_ref[...] = ub[:, :G2]
    b0_ref[...] = ub[:, G2:] + b_ref[0][None, :]


def _tc_nodes(xg, w, b):
    return pl.pallas_call(
        _tc_nodes_body,
        in_specs=[
            pl.BlockSpec((N, DF), lambda: (0, 0)),
            pl.BlockSpec((DF, 2 * G2), lambda: (0, 0)),
            pl.BlockSpec((1, G2), lambda: (0, 0)),
        ],
        out_specs=[
            pl.BlockSpec((N, G2), lambda: (0, 0)),
            pl.BlockSpec((N, G2), lambda: (0, 0)),
        ],
        out_shape=[
            jax.ShapeDtypeStruct((N, G2), jnp.float32),
            jax.ShapeDtypeStruct((N, G2), jnp.float32),
        ],
    )(xg, w, b)


EBLK8 = E8 // 5


def _tc_edges_body(xe_ref, w1_ref, b1_ref, w2_ref, b2_ref, v_ref):
    h = jnp.maximum(
        jnp.dot(xe_ref[...], w1_ref[...], preferred_element_type=jnp.float32)
        + b1_ref[0][None, :], 0.0)
    v_ref[...] = jnp.dot(h, w2_ref[...],
                         preferred_element_type=jnp.float32) + b2_ref[0]


def _tc_edges(xep, w1b, b1b, w2b, b2):
    return pl.pallas_call(
        _tc_edges_body,
        grid=(5,),
        in_specs=[
            pl.BlockSpec((EBLK8, 8 * DE), lambda j: (j, 0)),
            pl.BlockSpec((8 * DE, 8 * HD), lambda j: (0, 0)),
            pl.BlockSpec((1, 8 * HD), lambda j: (0, 0)),
            pl.BlockSpec((8 * HD, 8), lambda j: (0, 0)),
            pl.BlockSpec(memory_space=pltpu.SMEM),
        ],
        out_specs=pl.BlockSpec((EBLK8, 8), lambda j: (j, 0)),
        out_shape=jax.ShapeDtypeStruct((E8, 8), jnp.float32),
    )(xep, w1b, b1b, w2b, b2)


def _sc_body(idx_hbm, v_hbm, u_hbm, s_hbm,
             s_sh, deg_sh,
             drow_a, dew_a, drow_b, dew_b, row2, lw2, deg2d, zdeg, i128,
             *bufs):
    rows = (bufs[0:NBUF], bufs[NBUF:2 * NBUF])
    colc = (bufs[2 * NBUF:3 * NBUF], bufs[3 * NBUF:4 * NBUF])
    gsem = bufs[4 * NBUF:4 * NBUF + 2]
    ssem = bufs[4 * NBUF + 2:4 * NBUF + 4]
    cid = lax.axis_index("c")
    sid = lax.axis_index("s")
    wid = cid * NT + sid
    zf16 = jnp.zeros((16,), jnp.float32)

    def _z1(i, _):
        deg2d[i] = zf16
        return 0
    lax.fori_loop(0, NR, _z1, 0)

    def _zz(i, _):
        zdeg[i] = zf16
        return 0
    lax.fori_loop(0, NR // NT, _zz, 0)

    def _z2(i, _):
        for j in range(G2 // 16):
            rows[0][0][i, pl.ds(j * 16, 16)] = zf16
        return 0
    lax.fori_loop(0, C, _z2, 0)

    st1 = pltpu.async_copy(idx_hbm.at[0, pl.ds(wid * EPW, EPW)], row2,
                           gsem[0])
    st2 = pltpu.async_copy(v_hbm.at[pl.ds(wid * EPW, EPW)], lw2, gsem[0])

    zd = []
    for q in range(NPAD // NT // C):
        zd.append(pltpu.async_copy(
            rows[0][0], s_sh.at[pl.ds(sid * (NPAD // NT) + q * C, C)],
            ssem[0]))
    pltpu.sync_copy(zdeg, deg_sh.at[pl.ds(sid * (NR // NT), NR // NT)])
    for d in zd:
        d.wait()
    st1.wait()
    st2.wait()


    def _sigmoid16(xv):
        t = jnp.exp(-jnp.maximum(xv, -80.0)) + 1.0
        ib = jnp.int32(0x7EF311C3) - lax.bitcast_convert_type(t, jnp.int32)
        r = lax.bitcast_convert_type(ib, jnp.float32)
        r = r * (2.0 - t * r)
        r = r * (2.0 - t * r)
        r = r * (2.0 - t * r)
        return r

    def _sig(i, _):
        o = pl.ds(i * 16, 16)
        lw2[o] = _sigmoid16(lw2[o])
        return 0
    lax.fori_loop(0, EPW // 16, _sig, 0)

    NDCH = EPT // DCH

    def _dissue(q, rbuf, ebuf, sem):
        base = sid * EPT + q * DCH
        pltpu.async_copy(idx_hbm.at[0, pl.ds(base, DCH)], rbuf, sem)
        pltpu.async_copy(v_hbm.at[pl.ds(base, DCH)], ebuf, sem)

    def _ddrain(rbuf, ebuf, sem):
        pltpu.make_async_copy(idx_hbm.at[0, pl.ds(0, DCH)], rbuf, sem).wait()
        pltpu.make_async_copy(v_hbm.at[pl.ds(0, DCH)], ebuf, sem).wait()

    def _dacc_on(rbuf, ebuf):
        def _dacc(i, _):
            o = pl.ds(i * 16, 16)
            r = rbuf[o]
            e = _sigmoid16(ebuf[o])
            plsc.addupdate_scatter(
                deg2d, [lax.shift_right_logical(r, 4),
                        jnp.bitwise_and(r, 15)], e)
            return 0
        lax.fori_loop(0, DCH // 16, _dacc, 0)

    _dissue(0, drow_a, dew_a, gsem[0])
    def _dpair(p, _):
        _ddrain(drow_a, dew_a, gsem[0])
        _dissue(2 * p + 1, drow_b, dew_b, gsem[1])
        _dacc_on(drow_a, dew_a)
        _ddrain(drow_b, dew_b, gsem[1])

        @pl.when(2 * p + 2 < NDCH)
        def _():
            _dissue(2 * p + 2, drow_a, dew_a, gsem[0])
        _dacc_on(drow_b, dew_b)
        return 0
    lax.fori_loop(0, NDCH // 2, _dpair, 0)
    _ddrain(drow_a, dew_a, gsem[0])
    _dacc_on(drow_a, dew_a)

    plsc.subcore_barrier()
    iota16 = lax.iota(jnp.int32, 16)
    for q in range(NR // 128):
        for j in range(8):
            i128[pl.ds(j * 16, 16)] = iota16 + (q * 128 + j * 16)
        pltpu.sync_copy(deg2d.at[pl.ds(q * 128, 128)],
                        deg_sh.at[i128], add=True)
    plsc.subcore_barrier()
    pltpu.sync_copy(deg_sh, deg2d)

    def _dinv(i, _):
        d = deg2d[i]
        ib = jnp.int32(0x5F3759DF) - lax.shift_right_arithmetic(
            lax.bitcast_convert_type(d, jnp.int32), 1)
        y = lax.bitcast_convert_type(ib, jnp.float32)
        y = y * (1.5 - 0.5 * d * y * y)
        y = y * (1.5 - 0.5 * d * y * y)
        y = y * (1.5 - 0.5 * d * y * y)
        deg2d[i] = jnp.where(d > 0.0, y, 0.0)
        return 0
    lax.fori_loop(0, NR, _dinv, 0)

    def _gather(g, st, b):
        ch = (g * NBUF + b) * C
        pltpu.async_copy(u_hbm.at[row2.at[pl.ds(ch, C)]],
                         rows[st][b], gsem[st])
        pltpu.async_copy(idx_hbm.at[1, pl.ds(wid * EPW + ch, C)],
                         colc[st][b], gsem[st])

    def _gdrain(st, b):
        pltpu.make_async_copy(
            u_hbm.at[pl.ds(0, C)], rows[st][b], gsem[st]).wait()
        pltpu.make_async_copy(
            idx_hbm.at[0, pl.ds(0, C)], colc[st][b], gsem[st]).wait()

    def _sdrain(st, b):
        pltpu.make_async_copy(
            u_hbm.at[pl.ds(0, C)], rows[st][b], ssem[st]).wait()

    for b in range(NBUF):
        _gather(0, 0, b)

    def _process(g, st):
        for b in range(NBUF):
            _gdrain(st, b)

        @pl.when(g > 0)
        def _():
            for b in range(NBUF):
                _sdrain(1 - st, b)

        @pl.when(g + 1 < GRP)
        def _():
            for b in range(NBUF):
                _gather(g + 1, 1 - st, b)

        for b in range(NBUF):
            ch = g * NBUF + b

            def _scale(m, _):
                o16 = pl.ds(ch * C + m * 16, 16)
                r = row2[o16]
                c = colc[st][b][pl.ds(m * 16, 16)]
                e = lw2[o16]
                a = plsc.load_gather(
                    deg2d, [lax.shift_right_logical(r, 4),
                            jnp.bitwise_and(r, 15)])
                bb = plsc.load_gather(
                    deg2d, [lax.shift_right_logical(c, 4),
                            jnp.bitwise_and(c, 15)])
                lv = -(a * e) * bb - jnp.where(r == c, 1.0, 0.0)
                for t in range(16):
                    ei = m * 16 + t
                    for j in range(G2 // 16):
                        o = pl.ds(j * 16, 16)
                        rows[st][b][ei, o] = rows[st][b][ei, o] * lv[t]
                return 0
            lax.fori_loop(0, C // 16, _scale, 0)
            pltpu.async_copy(rows[st][b], s_sh.at[colc[st][b]],
                             ssem[st], add=True)

    def _grp(gp, _):
        _process(2 * gp, 0)
        _process(2 * gp + 1, 1)
        return 0
    lax.fori_loop(0, GRP // 2, _grp, 0)
    _process(GRP - 1, 0)
    for b in range(NBUF):
        _sdrain(0, b)

    plsc.subcore_barrier()
    pltpu.sync_copy(s_sh.at[pl.ds(sid * (NPAD // NT), NPAD // NT)],
                    s_hbm.at[cid, pl.ds(sid * (NPAD // NT), NPAD // NT)])


@functools.cache
def _make_sc_spmm():
    @functools.partial(
        pl.kernel,
        out_type=jax.ShapeDtypeStruct((2, NPAD, G2), jnp.float32),
        mesh=plsc.VectorSubcoreMesh(core_axis_name="c", subcore_axis_name="s"),
        compiler_params=pltpu.CompilerParams(
            needs_layout_passes=False, use_tc_tiling_on_sc=False),
        scratch_types=[
            pltpu.VMEM_SHARED((NPAD, G2), jnp.float32),
            pltpu.VMEM_SHARED((NR, 16), jnp.float32),
            pltpu.VMEM((DCH,), jnp.int32),
            pltpu.VMEM((DCH,), jnp.float32),
            pltpu.VMEM((DCH,), jnp.int32),
            pltpu.VMEM((DCH,), jnp.float32),
            pltpu.VMEM((EPW,), jnp.int32),
            pltpu.VMEM((EPW,), jnp.float32),
            pltpu.VMEM((NR, 16), jnp.float32),
            pltpu.VMEM((NR // NT, 16), jnp.float32),
            pltpu.VMEM((128,), jnp.int32),
        ] + [pltpu.VMEM((C, G2), jnp.float32)] * (2 * NBUF)
          + [pltpu.VMEM((C,), jnp.int32)] * (2 * NBUF)
          + [pltpu.SemaphoreType.DMA] * 4,
    )
    def _sc_spmm(idx_hbm, v_hbm, u_hbm, s_hbm, *rest):
        _sc_body(idx_hbm, v_hbm, u_hbm, s_hbm, *rest)
    return _sc_spmm


KR = N * HD // 128
KRP = NPAD * HD // 128


def _tc_post_body(bz_ref, bh_ref, s0z_ref, s0h_ref, s1z_ref, s1h_ref,
                  med_ref, key_ref):
    az = bz_ref[...] + s0z_ref[:KR] + s1z_ref[:KR]
    ah = bh_ref[...] + s0h_ref[:KR] + s1h_ref[:KR]
    tz = jnp.tanh(az * 0.5)
    o = (0.5 - 0.5 * tz) * jnp.tanh(ah)
    b = lax.bitcast_convert_type(o, jnp.uint32)
    key_ref[...] = jnp.where((b >> 31) != 0, ~b, b | jnp.uint32(0x80000000))

    kth = (N * HD - 1) // 2

    def _bit(i, carry):
        prefix, k = carry
        sh = 31 - i
        cnt0 = jnp.sum(((key_ref[...] >> sh) == (prefix >> sh))
                       .astype(jnp.int32))
        take1 = k >= cnt0
        prefix = jnp.where(take1, prefix | (jnp.uint32(1) << sh), prefix)
        k = jnp.where(take1, k - cnt0, k)
        return prefix, k

    prefix, _ = lax.fori_loop(
        0, 32, _bit, (jnp.uint32(0), jnp.int32(kth)))
    medbits = jnp.where(
        (prefix >> 31) != 0, prefix ^ jnp.uint32(0x80000000), ~prefix)
    med_ref[0] = lax.bitcast_convert_type(medbits, jnp.float32)


def _tc_post(b0, s):
    bz = b0[:, :HD].reshape(KR, 128)
    bh = b0[:, HD:].reshape(KR, 128)
    s0z = s[0, :, :HD].reshape(KRP, 128)
    s0h = s[0, :, HD:].reshape(KRP, 128)
    s1z = s[1, :, :HD].reshape(KRP, 128)
    s1h = s[1, :, HD:].reshape(KRP, 128)
    full = lambda r: pl.BlockSpec((r, 128), lambda: (0, 0))
    return pl.pallas_call(
        _tc_post_body,
        in_specs=[full(KR), full(KR), full(KRP), full(KRP),
                  full(KRP), full(KRP)],
        out_specs=pl.BlockSpec(memory_space=pltpu.SMEM),
        out_shape=jax.ShapeDtypeStruct((1,), jnp.float32),
        scratch_shapes=[pltpu.VMEM((KR, 128), jnp.uint32)],
    )(bz, bh, s0z, s0h, s1z, s1h)


def kernel(x, xi, xe, y, yi, ye, p1, p2):
    eye8 = jnp.eye(8, dtype=jnp.float32)
    meds = []
    for xg, ig, xeg, p in ((x, xi, xe, p1), (y, yi, ye, p2)):
        w = jnp.concatenate(
            [p["xz_w1"], p["xh_w1"], p["xz_w0"], p["xh_w0"]], 1)
        b = jnp.concatenate(
            [p["xz_b"] + p["hz_b"], p["xh_b"] + p["hh_b"]])[None, :]
        u, b0 = _tc_nodes(xg, w, b)
        v = _tc_edges(
            xeg.reshape(E8, 8 * DE),
            jnp.kron(eye8, p["seq_w1"]),
            jnp.tile(p["seq_b1"], 8)[None, :],
            jnp.kron(eye8, p["seq_w2"]),
            p["seq_b2"])
        s = _make_sc_spmm()(ig, v.reshape(E), u)
        meds.append(_tc_post(b0, s))
    return jax.nn.sigmoid(meds[0][0] - meds[1][0])
